# Initial kernel scaffold; baseline (speedup 1.0000x reference)
import jax
import jax.numpy as jnp
from jax.experimental import pallas as pl


def kernel(H, W, rel_height, rel_width):
    raise NotImplementedError("write your pallas kernel here")


# speedup vs baseline: 5.6463x; 5.6463x over previous
import functools

import jax
import jax.numpy as jnp
from jax import lax
from jax.experimental import pallas as pl
from jax.experimental.pallas import tpu as pltpu
from jax.experimental.pallas import tpu_sc as plsc


def _sc_gather(rel_height, rel_width, dim, Hs, Ws):
    nc = 2
    ns = 16
    lanes = 16
    mesh = plsc.VectorSubcoreMesh(core_axis_name="c", subcore_axis_name="s")

    @functools.partial(
        pl.kernel,
        mesh=mesh,
        compiler_params=pltpu.CompilerParams(needs_layout_passes=False),
        out_type=(
            jax.ShapeDtypeStruct((Hs, dim, Hs), jnp.float32),
            jax.ShapeDtypeStruct((Ws, dim, Ws), jnp.float32),
        ),
        scratch_types=[
            pltpu.VMEM((2 * Hs - 1, dim), jnp.float32),
            pltpu.VMEM((2 * Ws - 1, dim), jnp.float32),
            pltpu.VMEM((dim, Hs), jnp.float32),
            pltpu.VMEM((dim, Ws), jnp.float32),
        ],
    )
    def gather_kernel(rh_hbm, rw_hbm, eh_hbm, ew_hbm, rh_v, rw_v, ehs, ews):
        wid = lax.axis_index("s") * nc + lax.axis_index("c")
        pltpu.sync_copy(rh_hbm, rh_v)
        pltpu.sync_copy(rw_hbm, rw_v)
        lane = lax.iota(jnp.int32, lanes)
        for c in range(Hs // lanes):
            r = (Hs - 1 + wid - lanes * c) - lane
            for d in range(dim):
                dv = jnp.full((lanes,), d, jnp.int32)
                ehs[d, pl.ds(lanes * c, lanes)] = plsc.load_gather(rh_v, [r, dv])
                ews[d, pl.ds(lanes * c, lanes)] = plsc.load_gather(rw_v, [r, dv])
        pltpu.sync_copy(ehs, eh_hbm.at[wid])
        pltpu.sync_copy(ews, ew_hbm.at[wid])

    return gather_kernel(rel_height, rel_width)


def _tc_expand(eh3, ew3, dim, Hs, Ws):
    HW = Hs * Ws

    def body(eh_ref, ew_ref, out_ref):
        ehm = eh_ref[0]
        ewm = ew_ref[0]
        colh = lax.broadcasted_iota(jnp.int32, (Hs, HW), 1) // Ws
        rowh = lax.broadcasted_iota(jnp.int32, (Hs, HW), 0)
        colw = lax.broadcasted_iota(jnp.int32, (Ws, HW), 1) % Ws
        roww = lax.broadcasted_iota(jnp.int32, (Ws, HW), 0)
        PT = (colh == rowh).astype(jnp.float32)
        QT = (colw == roww).astype(jnp.float32)
        EHb = jnp.dot(ehm, PT, preferred_element_type=jnp.float32)
        EWb = jnp.dot(ewm, QT, preferred_element_type=jnp.float32)
        for i in range(Hs):
            out_ref[0, pl.ds(i * Ws, Ws), :] = EHb[i:i + 1, :] + EWb

    return pl.pallas_call(
        body,
        grid=(dim,),
        in_specs=[
            pl.BlockSpec((1, Hs, Hs), lambda d: (d, 0, 0)),
            pl.BlockSpec((1, Ws, Ws), lambda d: (d, 0, 0)),
        ],
        out_specs=pl.BlockSpec((1, HW, HW), lambda d: (d, 0, 0)),
        out_shape=jax.ShapeDtypeStruct((dim, HW, HW), jnp.float32),
    )(eh3, ew3)


def kernel(H, W, rel_height, rel_width):
    del H, W
    dim = rel_height.shape[1]
    Hs = (rel_height.shape[0] + 1) // 2
    Ws = (rel_width.shape[0] + 1) // 2
    eh_sc, ew_sc = _sc_gather(rel_height, rel_width, dim, Hs, Ws)
    eh3 = jnp.transpose(eh_sc, (1, 0, 2))
    ew3 = jnp.transpose(ew_sc, (1, 0, 2))
    out = _tc_expand(eh3, ew3, dim, Hs, Ws)
    return out[None]
